# Initial kernel scaffold; baseline (speedup 1.0000x reference)
import jax
import jax.numpy as jnp
from jax.experimental import pallas as pl


def kernel(pretrained, finetuned):
    raise NotImplementedError("write your pallas kernel here")


# speedup vs baseline: 15.1484x; 15.1484x over previous
import functools

import jax
import jax.numpy as jnp
from jax import lax
from jax.experimental import pallas as pl
from jax.experimental.pallas import tpu as pltpu
from jax.experimental.pallas import tpu_sc as plsc

_N = 16777216
_K = _N // 100
_NTILES = 32
_PER_TILE = _N // _NTILES
_CHUNK = 8192
_NCHUNKS = _PER_TILE // _CHUNK
_VECS = _CHUNK // 16
_L1B = 2048
_L2B = 1024
_L3B = 1024
_SIG_HI = 0.9933071490757153
_SIG_LO = 0.006692850924284856

_mesh = plsc.VectorSubcoreMesh(core_axis_name="c", subcore_axis_name="s")
_params = pltpu.CompilerParams(needs_layout_passes=False)


def _wid():
    return lax.axis_index("s") * 2 + lax.axis_index("c")


def _zero_vmem(ref, nwords):
    z = jnp.zeros((16,), jnp.int32)

    def body(i, c):
        ref[pl.ds(i * 16, 16)] = z
        return c

    lax.fori_loop(0, nwords // 16, body, 0)


def _reduce_lanes(histv, houtv, nbins):
    def body(j, c):
        acc = histv[pl.ds(j * 16, 16)]
        for l in range(1, 16):
            acc = acc + histv[pl.ds(l * nbins + j * 16, 16)]
        houtv[pl.ds(j * 16, 16)] = acc
        return c

    lax.fori_loop(0, nbins // 16, body, 0)


@functools.partial(
    pl.kernel,
    mesh=_mesh,
    compiler_params=_params,
    out_type=[
        jax.ShapeDtypeStruct((_N,), jnp.float32),
        jax.ShapeDtypeStruct((_NTILES, _L1B), jnp.int32),
    ],
    scratch_types=[
        pltpu.VMEM((_CHUNK,), jnp.float32),
        pltpu.VMEM((_CHUNK,), jnp.float32),
        pltpu.VMEM((_CHUNK,), jnp.float32),
        pltpu.VMEM((16 * _L1B,), jnp.int32),
        pltpu.VMEM((_L1B,), jnp.int32),
    ],
)
def _pass1(p_hbm, f_hbm, tv_hbm, hist_hbm, pbuf, fbuf, tvbuf, histv, houtv):
    wid = _wid()
    base = wid * _PER_TILE
    lane_base = lax.iota(jnp.int32, 16) * _L1B
    ones = jnp.ones((16,), jnp.int32)
    _zero_vmem(histv, 16 * _L1B)

    def chunk(ci, c):
        off = base + ci * _CHUNK
        pltpu.sync_copy(p_hbm.at[pl.ds(off, _CHUNK)], pbuf)
        pltpu.sync_copy(f_hbm.at[pl.ds(off, _CHUNK)], fbuf)

        def vec(i, cc):
            tv = fbuf[pl.ds(i * 16, 16)] - pbuf[pl.ds(i * 16, 16)]
            tvbuf[pl.ds(i * 16, 16)] = tv
            u = plsc.bitcast(tv, jnp.int32) & jnp.int32(0x7FFFFFFF)
            idx = lane_base + (u >> 20)
            plsc.addupdate_scatter(histv, [idx], ones)
            return cc

        lax.fori_loop(0, _VECS, vec, 0)
        pltpu.sync_copy(tvbuf, tv_hbm.at[pl.ds(off, _CHUNK)])
        return c

    lax.fori_loop(0, _NCHUNKS, chunk, 0)
    _reduce_lanes(histv, houtv, _L1B)
    pltpu.sync_copy(houtv, hist_hbm.at[wid])


def _masked_hist_kernel(shift_mask, shift_bucket, nbins):
    @functools.partial(
        pl.kernel,
        mesh=_mesh,
        compiler_params=_params,
        out_type=jax.ShapeDtypeStruct((_NTILES, nbins), jnp.int32),
        scratch_types=[
            pltpu.VMEM((_CHUNK,), jnp.float32),
            pltpu.VMEM((16,), jnp.int32),
            pltpu.VMEM((16 * nbins,), jnp.int32),
            pltpu.VMEM((nbins,), jnp.int32),
        ],
    )
    def _pass(tv_hbm, sel_hbm, hist_hbm, tvbuf, selv, histv, houtv):
        wid = _wid()
        base = wid * _PER_TILE
        lane_base = lax.iota(jnp.int32, 16) * nbins
        ones = jnp.ones((16,), jnp.int32)
        _zero_vmem(histv, 16 * nbins)
        pltpu.sync_copy(sel_hbm, selv)
        prefix = selv[...]

        def chunk(ci, c):
            off = base + ci * _CHUNK
            pltpu.sync_copy(tv_hbm.at[pl.ds(off, _CHUNK)], tvbuf)

            def vec(i, cc):
                tv = tvbuf[pl.ds(i * 16, 16)]
                u = plsc.bitcast(tv, jnp.int32) & jnp.int32(0x7FFFFFFF)
                mask = (u >> shift_mask) == prefix
                idx = lane_base + ((u >> shift_bucket) & jnp.int32(nbins - 1))
                plsc.addupdate_scatter(histv, [idx], ones, mask=mask)
                return cc

            lax.fori_loop(0, _VECS, vec, 0)
            return c

        lax.fori_loop(0, _NCHUNKS, chunk, 0)
        _reduce_lanes(histv, houtv, nbins)
        pltpu.sync_copy(houtv, hist_hbm.at[wid])

    return _pass


_pass2 = _masked_hist_kernel(20, 10, _L2B)
_pass3 = _masked_hist_kernel(10, 0, _L3B)


@functools.partial(
    pl.kernel,
    mesh=_mesh,
    compiler_params=_params,
    out_type=jax.ShapeDtypeStruct((_N,), jnp.float32),
    scratch_types=[
        pltpu.VMEM((_CHUNK,), jnp.float32),
        pltpu.VMEM((16,), jnp.int32),
        pltpu.VMEM((_CHUNK,), jnp.float32),
    ],
)
def _pass4(tv_hbm, thr_hbm, out_hbm, tvbuf, thrv, outbuf):
    wid = _wid()
    base = wid * _PER_TILE
    hi = jnp.full((16,), _SIG_HI, jnp.float32)
    lo = jnp.full((16,), _SIG_LO, jnp.float32)
    pltpu.sync_copy(thr_hbm, thrv)
    uthr = thrv[...]

    def chunk(ci, c):
        off = base + ci * _CHUNK
        pltpu.sync_copy(tv_hbm.at[pl.ds(off, _CHUNK)], tvbuf)

        def vec(i, cc):
            tv = tvbuf[pl.ds(i * 16, 16)]
            u = plsc.bitcast(tv, jnp.int32) & jnp.int32(0x7FFFFFFF)
            frac = jnp.where(u > uthr, hi, lo)
            outbuf[pl.ds(i * 16, 16)] = tv * frac
            return cc

        lax.fori_loop(0, _VECS, vec, 0)
        pltpu.sync_copy(outbuf, out_hbm.at[pl.ds(off, _CHUNK)])
        return c

    lax.fori_loop(0, _NCHUNKS, chunk, 0)


def _select(hist, krem):
    tail = jnp.cumsum(hist[::-1])[::-1]
    b = jnp.sum((tail >= krem).astype(jnp.int32)) - 1
    above = tail[b] - hist[b]
    return b, krem - above


def kernel(pretrained, finetuned):
    tv, h1_tiles = _pass1(pretrained, finetuned)
    h1 = jnp.sum(h1_tiles, axis=0)
    b1, k1 = _select(h1, jnp.int32(_K))

    sel1 = jnp.full((16,), b1, jnp.int32)
    h2_tiles = _pass2(tv, sel1)
    h2 = jnp.sum(h2_tiles, axis=0)
    b2, k2 = _select(h2, k1)

    sel2 = jnp.full((16,), (b1 << 10) | b2, jnp.int32)
    h3_tiles = _pass3(tv, sel2)
    h3 = jnp.sum(h3_tiles, axis=0)
    b3, k3 = _select(h3, k2)

    uthr = (b1 << 20) | (b2 << 10) | b3
    thr_vec = jnp.full((16,), uthr, jnp.int32)
    masked_delta = _pass4(tv, thr_vec)

    cnt_gt = jnp.int32(_K) - k3
    prop = 5.0 * (2.0 * cnt_gt.astype(jnp.float32) - float(_N)) / float(_N)
    return masked_delta, prop


# speedup vs baseline: 22.8001x; 1.5051x over previous
import functools

import jax
import jax.numpy as jnp
from jax import lax
from jax.experimental import pallas as pl
from jax.experimental.pallas import tpu as pltpu
from jax.experimental.pallas import tpu_sc as plsc

_N = 16777216
_K = _N // 100
_NTILES = 32
_PER_TILE = _N // _NTILES
_L1B = 2048
_L2B = 1024
_L3B = 1024
_UNROLL = 8
_SIG_HI = 0.9933071490757153
_SIG_LO = 0.006692850924284856

_mesh = plsc.VectorSubcoreMesh(core_axis_name="c", subcore_axis_name="s")
_params = pltpu.CompilerParams(needs_layout_passes=False)


def _wid():
    return lax.axis_index("s") * 2 + lax.axis_index("c")


def _zero_vmem(ref, nwords):
    z = jnp.zeros((16,), jnp.int32)

    def body(i, c):
        for u in range(_UNROLL):
            ref[pl.ds(i * 16 * _UNROLL + u * 16, 16)] = z
        return c

    lax.fori_loop(0, nwords // (16 * _UNROLL), body, 0)


def _reduce_lanes(histv, houtv, nbins):
    def body(j, c):
        acc = histv[pl.ds(j * 16, 16)]
        for l in range(1, 16):
            acc = acc + histv[pl.ds(l * nbins + j * 16, 16)]
        houtv[pl.ds(j * 16, 16)] = acc
        return c

    lax.fori_loop(0, nbins // 16, body, 0)


def _pipeline(base, chunk, nchunks, ins, outs, compute):
    def start_in(ci, b):
        off = base + ci * chunk
        for hbm, bufs, sems in ins:
            pltpu.async_copy(hbm.at[pl.ds(off, chunk)], bufs[b], sems[b])

    for b in range(2):
        start_in(b, b)

    def pair(cp, c):
        for b in range(2):
            ci = cp * 2 + b
            for hbm, bufs, sems in ins:
                pltpu.make_async_copy(
                    hbm.at[pl.ds(0, chunk)], bufs[b], sems[b]
                ).wait()

            @pl.when(cp > 0)
            def _():
                for hbm, bufs, sems in outs:
                    pltpu.make_async_copy(
                        bufs[b], hbm.at[pl.ds(0, chunk)], sems[b]
                    ).wait()

            compute(b, ci)
            off = base + ci * chunk
            for hbm, bufs, sems in outs:
                pltpu.async_copy(bufs[b], hbm.at[pl.ds(off, chunk)], sems[b])

            @pl.when(ci + 2 < nchunks)
            def _():
                start_in(ci + 2, b)
        return c

    lax.fori_loop(0, nchunks // 2, pair, 0)
    for b in range(2):
        for hbm, bufs, sems in outs:
            pltpu.make_async_copy(bufs[b], hbm.at[pl.ds(0, chunk)], sems[b]).wait()


_P1_CHUNK = 8192


@functools.partial(
    pl.kernel,
    mesh=_mesh,
    compiler_params=_params,
    out_type=[
        jax.ShapeDtypeStruct((_N,), jnp.float32),
        jax.ShapeDtypeStruct((_NTILES, _L1B), jnp.int32),
    ],
    scratch_types=[
        pltpu.VMEM((_P1_CHUNK,), jnp.float32),
        pltpu.VMEM((_P1_CHUNK,), jnp.float32),
        pltpu.VMEM((_P1_CHUNK,), jnp.float32),
        pltpu.VMEM((_P1_CHUNK,), jnp.float32),
        pltpu.VMEM((_P1_CHUNK,), jnp.float32),
        pltpu.VMEM((_P1_CHUNK,), jnp.float32),
        pltpu.VMEM((16 * _L1B,), jnp.int32),
        pltpu.VMEM((_L1B,), jnp.int32),
        pltpu.SemaphoreType.DMA,
        pltpu.SemaphoreType.DMA,
        pltpu.SemaphoreType.DMA,
        pltpu.SemaphoreType.DMA,
        pltpu.SemaphoreType.DMA,
        pltpu.SemaphoreType.DMA,
    ],
)
def _pass1(
    p_hbm, f_hbm, tv_hbm, hist_hbm,
    pb0, pb1, fb0, fb1, tb0, tb1, histv, houtv,
    sp0, sp1, sf0, sf1, so0, so1,
):
    wid = _wid()
    base = wid * _PER_TILE
    lane_base = lax.iota(jnp.int32, 16) * _L1B
    ones = jnp.ones((16,), jnp.int32)
    _zero_vmem(histv, 16 * _L1B)
    pbufs, fbufs, tbufs = [pb0, pb1], [fb0, fb1], [tb0, tb1]

    def compute(b, ci):
        pbuf, fbuf, tbuf = pbufs[b], fbufs[b], tbufs[b]

        def vec(i, cc):
            for u in range(_UNROLL):
                s = i * 16 * _UNROLL + u * 16
                tvv = fbuf[pl.ds(s, 16)] - pbuf[pl.ds(s, 16)]
                tbuf[pl.ds(s, 16)] = tvv
                w = plsc.bitcast(tvv, jnp.int32) & jnp.int32(0x7FFFFFFF)
                plsc.addupdate_scatter(histv, [lane_base + (w >> 20)], ones)
            return cc

        lax.fori_loop(0, _P1_CHUNK // (16 * _UNROLL), vec, 0)

    _pipeline(
        base, _P1_CHUNK, _PER_TILE // _P1_CHUNK,
        [(p_hbm, pbufs, [sp0, sp1]), (f_hbm, fbufs, [sf0, sf1])],
        [(tv_hbm, tbufs, [so0, so1])],
        compute,
    )
    _reduce_lanes(histv, houtv, _L1B)
    pltpu.sync_copy(houtv, hist_hbm.at[wid])


_PH_CHUNK = 32768


def _masked_hist_kernel(shift_mask, shift_bucket, nbins):
    @functools.partial(
        pl.kernel,
        mesh=_mesh,
        compiler_params=_params,
        out_type=jax.ShapeDtypeStruct((_NTILES, nbins), jnp.int32),
        scratch_types=[
            pltpu.VMEM((_PH_CHUNK,), jnp.float32),
            pltpu.VMEM((_PH_CHUNK,), jnp.float32),
            pltpu.VMEM((16,), jnp.int32),
            pltpu.VMEM((16 * nbins,), jnp.int32),
            pltpu.VMEM((nbins,), jnp.int32),
            pltpu.SemaphoreType.DMA,
            pltpu.SemaphoreType.DMA,
        ],
    )
    def _pass(tv_hbm, sel_hbm, hist_hbm, tb0, tb1, selv, histv, houtv, s0, s1):
        wid = _wid()
        base = wid * _PER_TILE
        lane_base = lax.iota(jnp.int32, 16) * nbins
        ones = jnp.ones((16,), jnp.int32)
        _zero_vmem(histv, 16 * nbins)
        pltpu.sync_copy(sel_hbm, selv)
        prefix = selv[...]
        tbufs = [tb0, tb1]

        def compute(b, ci):
            tbuf = tbufs[b]

            def vec(i, cc):
                for u in range(_UNROLL):
                    s = i * 16 * _UNROLL + u * 16
                    w = plsc.bitcast(tbuf[pl.ds(s, 16)], jnp.int32) & jnp.int32(
                        0x7FFFFFFF
                    )
                    mask = (w >> shift_mask) == prefix
                    idx = lane_base + ((w >> shift_bucket) & jnp.int32(nbins - 1))
                    plsc.addupdate_scatter(histv, [idx], ones, mask=mask)
                return cc

            lax.fori_loop(0, _PH_CHUNK // (16 * _UNROLL), vec, 0)

        _pipeline(
            base, _PH_CHUNK, _PER_TILE // _PH_CHUNK,
            [(tv_hbm, tbufs, [s0, s1])], [], compute,
        )
        _reduce_lanes(histv, houtv, nbins)
        pltpu.sync_copy(houtv, hist_hbm.at[wid])

    return _pass


_pass2 = _masked_hist_kernel(20, 10, _L2B)
_pass3 = _masked_hist_kernel(10, 0, _L3B)


_P4_CHUNK = 16384


@functools.partial(
    pl.kernel,
    mesh=_mesh,
    compiler_params=_params,
    out_type=jax.ShapeDtypeStruct((_N,), jnp.float32),
    scratch_types=[
        pltpu.VMEM((_P4_CHUNK,), jnp.float32),
        pltpu.VMEM((_P4_CHUNK,), jnp.float32),
        pltpu.VMEM((_P4_CHUNK,), jnp.float32),
        pltpu.VMEM((_P4_CHUNK,), jnp.float32),
        pltpu.VMEM((16,), jnp.int32),
        pltpu.SemaphoreType.DMA,
        pltpu.SemaphoreType.DMA,
        pltpu.SemaphoreType.DMA,
        pltpu.SemaphoreType.DMA,
    ],
)
def _pass4(tv_hbm, thr_hbm, out_hbm, tb0, tb1, ob0, ob1, thrv, si0, si1, so0, so1):
    wid = _wid()
    base = wid * _PER_TILE
    hi = jnp.full((16,), _SIG_HI, jnp.float32)
    lo = jnp.full((16,), _SIG_LO, jnp.float32)
    pltpu.sync_copy(thr_hbm, thrv)
    uthr = thrv[...]
    tbufs, obufs = [tb0, tb1], [ob0, ob1]

    def compute(b, ci):
        tbuf, obuf = tbufs[b], obufs[b]

        def vec(i, cc):
            for u in range(_UNROLL):
                s = i * 16 * _UNROLL + u * 16
                tvv = tbuf[pl.ds(s, 16)]
                w = plsc.bitcast(tvv, jnp.int32) & jnp.int32(0x7FFFFFFF)
                obuf[pl.ds(s, 16)] = tvv * jnp.where(w > uthr, hi, lo)
            return cc

        lax.fori_loop(0, _P4_CHUNK // (16 * _UNROLL), vec, 0)

    _pipeline(
        base, _P4_CHUNK, _PER_TILE // _P4_CHUNK,
        [(tv_hbm, tbufs, [si0, si1])],
        [(out_hbm, obufs, [so0, so1])],
        compute,
    )


def _select(hist, krem):
    tail = jnp.cumsum(hist[::-1])[::-1]
    b = jnp.sum((tail >= krem).astype(jnp.int32)) - 1
    above = tail[b] - hist[b]
    return b, krem - above


def kernel(pretrained, finetuned):
    tv, h1_tiles = _pass1(pretrained, finetuned)
    h1 = jnp.sum(h1_tiles, axis=0)
    b1, k1 = _select(h1, jnp.int32(_K))

    sel1 = jnp.full((16,), b1, jnp.int32)
    h2_tiles = _pass2(tv, sel1)
    h2 = jnp.sum(h2_tiles, axis=0)
    b2, k2 = _select(h2, k1)

    sel2 = jnp.full((16,), (b1 << 10) | b2, jnp.int32)
    h3_tiles = _pass3(tv, sel2)
    h3 = jnp.sum(h3_tiles, axis=0)
    b3, k3 = _select(h3, k2)

    uthr = (b1 << 20) | (b2 << 10) | b3
    thr_vec = jnp.full((16,), uthr, jnp.int32)
    masked_delta = _pass4(tv, thr_vec)

    cnt_gt = jnp.int32(_K) - k3
    prop = 5.0 * (2.0 * cnt_gt.astype(jnp.float32) - float(_N)) / float(_N)
    return masked_delta, prop


# speedup vs baseline: 24.1222x; 1.0580x over previous
import functools

import jax
import jax.numpy as jnp
from jax import lax
from jax.experimental import pallas as pl
from jax.experimental.pallas import tpu as pltpu
from jax.experimental.pallas import tpu_sc as plsc

_N = 16777216
_K = _N // 100
_NTILES = 32
_PER_TILE = _N // _NTILES
_L1B = 2048
_L2B = 1024
_L3B = 1024
_UNROLL = 8
_SIG_HI = 0.9933071490757153
_SIG_LO = 0.006692850924284856

_mesh = plsc.VectorSubcoreMesh(core_axis_name="c", subcore_axis_name="s")
_params = pltpu.CompilerParams(needs_layout_passes=False)


def _wid():
    return lax.axis_index("s") * 2 + lax.axis_index("c")


def _zero_vmem(ref, nwords):
    z = jnp.zeros((16,), jnp.int32)

    def body(i, c):
        for u in range(_UNROLL):
            ref[pl.ds(i * 16 * _UNROLL + u * 16, 16)] = z
        return c

    lax.fori_loop(0, nwords // (16 * _UNROLL), body, 0)


def _pipeline(base, chunk, nchunks, ins, outs, compute):
    def start_in(ci, b):
        off = base + ci * chunk
        for hbm, bufs, sems in ins:
            pltpu.async_copy(hbm.at[pl.ds(off, chunk)], bufs[b], sems[b])

    for b in range(2):
        start_in(b, b)

    def pair(cp, c):
        for b in range(2):
            ci = cp * 2 + b
            for hbm, bufs, sems in ins:
                pltpu.make_async_copy(
                    hbm.at[pl.ds(0, chunk)], bufs[b], sems[b]
                ).wait()

            @pl.when(cp > 0)
            def _():
                for hbm, bufs, sems in outs:
                    pltpu.make_async_copy(
                        bufs[b], hbm.at[pl.ds(0, chunk)], sems[b]
                    ).wait()

            compute(b, ci)
            off = base + ci * chunk
            for hbm, bufs, sems in outs:
                pltpu.async_copy(bufs[b], hbm.at[pl.ds(off, chunk)], sems[b])

            @pl.when(ci + 2 < nchunks)
            def _():
                start_in(ci + 2, b)
        return c

    lax.fori_loop(0, nchunks // 2, pair, 0)
    for b in range(2):
        for hbm, bufs, sems in outs:
            pltpu.make_async_copy(bufs[b], hbm.at[pl.ds(0, chunk)], sems[b]).wait()


_P1_CHUNK = 8192


@functools.partial(
    pl.kernel,
    mesh=_mesh,
    compiler_params=_params,
    out_type=[
        jax.ShapeDtypeStruct((_N,), jnp.float32),
        jax.ShapeDtypeStruct((_NTILES, 16 * _L1B), jnp.int32),
    ],
    scratch_types=[
        pltpu.VMEM((_P1_CHUNK,), jnp.float32),
        pltpu.VMEM((_P1_CHUNK,), jnp.float32),
        pltpu.VMEM((_P1_CHUNK,), jnp.float32),
        pltpu.VMEM((_P1_CHUNK,), jnp.float32),
        pltpu.VMEM((_P1_CHUNK,), jnp.float32),
        pltpu.VMEM((_P1_CHUNK,), jnp.float32),
        pltpu.VMEM((16 * _L1B,), jnp.int32),
        pltpu.SemaphoreType.DMA,
        pltpu.SemaphoreType.DMA,
        pltpu.SemaphoreType.DMA,
        pltpu.SemaphoreType.DMA,
        pltpu.SemaphoreType.DMA,
        pltpu.SemaphoreType.DMA,
    ],
)
def _pass1(
    p_hbm, f_hbm, tv_hbm, hist_hbm,
    pb0, pb1, fb0, fb1, tb0, tb1, histv,
    sp0, sp1, sf0, sf1, so0, so1,
):
    wid = _wid()
    base = wid * _PER_TILE
    lane = lax.iota(jnp.int32, 16)
    ones = jnp.ones((16,), jnp.int32)
    _zero_vmem(histv, 16 * _L1B)
    pbufs, fbufs, tbufs = [pb0, pb1], [fb0, fb1], [tb0, tb1]

    def compute(b, ci):
        pbuf, fbuf, tbuf = pbufs[b], fbufs[b], tbufs[b]

        def vec(i, cc):
            for u in range(_UNROLL):
                s = i * 16 * _UNROLL + u * 16
                tvv = fbuf[pl.ds(s, 16)] - pbuf[pl.ds(s, 16)]
                tbuf[pl.ds(s, 16)] = tvv
                w = plsc.bitcast(tvv, jnp.int32) & jnp.int32(0x7FFFFFFF)
                plsc.addupdate_scatter(histv, [((w >> 20) << 4) + lane], ones)
            return cc

        lax.fori_loop(0, _P1_CHUNK // (16 * _UNROLL), vec, 0)

    _pipeline(
        base, _P1_CHUNK, _PER_TILE // _P1_CHUNK,
        [(p_hbm, pbufs, [sp0, sp1]), (f_hbm, fbufs, [sf0, sf1])],
        [(tv_hbm, tbufs, [so0, so1])],
        compute,
    )
    pltpu.sync_copy(histv, hist_hbm.at[wid])


_PH_CHUNK = 32768


def _masked_hist_kernel(shift_mask, shift_bucket, nbins):
    @functools.partial(
        pl.kernel,
        mesh=_mesh,
        compiler_params=_params,
        out_type=jax.ShapeDtypeStruct((_NTILES, 16 * nbins), jnp.int32),
        scratch_types=[
            pltpu.VMEM((_PH_CHUNK,), jnp.float32),
            pltpu.VMEM((_PH_CHUNK,), jnp.float32),
            pltpu.VMEM((16,), jnp.int32),
            pltpu.VMEM((16 * nbins,), jnp.int32),
            pltpu.SemaphoreType.DMA,
            pltpu.SemaphoreType.DMA,
        ],
    )
    def _pass(tv_hbm, sel_hbm, hist_hbm, tb0, tb1, selv, histv, s0, s1):
        wid = _wid()
        base = wid * _PER_TILE
        lane = lax.iota(jnp.int32, 16)
        ones = jnp.ones((16,), jnp.int32)
        _zero_vmem(histv, 16 * nbins)
        pltpu.sync_copy(sel_hbm, selv)
        prefix = selv[...]
        tbufs = [tb0, tb1]

        def compute(b, ci):
            tbuf = tbufs[b]

            def vec(i, cc):
                for u in range(_UNROLL):
                    s = i * 16 * _UNROLL + u * 16
                    w = plsc.bitcast(tbuf[pl.ds(s, 16)], jnp.int32) & jnp.int32(
                        0x7FFFFFFF
                    )
                    mask = (w >> shift_mask) == prefix
                    idx = (((w >> shift_bucket) & jnp.int32(nbins - 1)) << 4) + lane
                    plsc.addupdate_scatter(histv, [idx], ones, mask=mask)
                return cc

            lax.fori_loop(0, _PH_CHUNK // (16 * _UNROLL), vec, 0)

        _pipeline(
            base, _PH_CHUNK, _PER_TILE // _PH_CHUNK,
            [(tv_hbm, tbufs, [s0, s1])], [], compute,
        )
        pltpu.sync_copy(histv, hist_hbm.at[wid])

    return _pass


_pass2 = _masked_hist_kernel(20, 10, _L2B)
_pass3 = _masked_hist_kernel(10, 0, _L3B)


_P4_CHUNK = 16384


@functools.partial(
    pl.kernel,
    mesh=_mesh,
    compiler_params=_params,
    out_type=jax.ShapeDtypeStruct((_N,), jnp.float32),
    scratch_types=[
        pltpu.VMEM((_P4_CHUNK,), jnp.float32),
        pltpu.VMEM((_P4_CHUNK,), jnp.float32),
        pltpu.VMEM((_P4_CHUNK,), jnp.float32),
        pltpu.VMEM((_P4_CHUNK,), jnp.float32),
        pltpu.VMEM((16,), jnp.int32),
        pltpu.SemaphoreType.DMA,
        pltpu.SemaphoreType.DMA,
        pltpu.SemaphoreType.DMA,
        pltpu.SemaphoreType.DMA,
    ],
)
def _pass4(tv_hbm, thr_hbm, out_hbm, tb0, tb1, ob0, ob1, thrv, si0, si1, so0, so1):
    wid = _wid()
    base = wid * _PER_TILE
    hi = jnp.full((16,), _SIG_HI, jnp.float32)
    lo = jnp.full((16,), _SIG_LO, jnp.float32)
    pltpu.sync_copy(thr_hbm, thrv)
    uthr = thrv[...]
    tbufs, obufs = [tb0, tb1], [ob0, ob1]

    def compute(b, ci):
        tbuf, obuf = tbufs[b], obufs[b]

        def vec(i, cc):
            for u in range(_UNROLL):
                s = i * 16 * _UNROLL + u * 16
                tvv = tbuf[pl.ds(s, 16)]
                w = plsc.bitcast(tvv, jnp.int32) & jnp.int32(0x7FFFFFFF)
                obuf[pl.ds(s, 16)] = tvv * jnp.where(w > uthr, hi, lo)
            return cc

        lax.fori_loop(0, _P4_CHUNK // (16 * _UNROLL), vec, 0)

    _pipeline(
        base, _P4_CHUNK, _PER_TILE // _P4_CHUNK,
        [(tv_hbm, tbufs, [si0, si1])],
        [(out_hbm, obufs, [so0, so1])],
        compute,
    )


def _select(hist, krem):
    tail = jnp.cumsum(hist[::-1])[::-1]
    b = jnp.sum((tail >= krem).astype(jnp.int32)) - 1
    above = tail[b] - hist[b]
    return b, krem - above


def kernel(pretrained, finetuned):
    tv, h1_tiles = _pass1(pretrained, finetuned)
    h1 = jnp.sum(h1_tiles.reshape(_NTILES, _L1B, 16), axis=(0, 2))
    b1, k1 = _select(h1, jnp.int32(_K))

    sel1 = jnp.full((16,), b1, jnp.int32)
    h2_tiles = _pass2(tv, sel1)
    h2 = jnp.sum(h2_tiles.reshape(_NTILES, _L2B, 16), axis=(0, 2))
    b2, k2 = _select(h2, k1)

    sel2 = jnp.full((16,), (b1 << 10) | b2, jnp.int32)
    h3_tiles = _pass3(tv, sel2)
    h3 = jnp.sum(h3_tiles.reshape(_NTILES, _L3B, 16), axis=(0, 2))
    b3, k3 = _select(h3, k2)

    uthr = (b1 << 20) | (b2 << 10) | b3
    thr_vec = jnp.full((16,), uthr, jnp.int32)
    masked_delta = _pass4(tv, thr_vec)

    cnt_gt = jnp.int32(_K) - k3
    prop = 5.0 * (2.0 * cnt_gt.astype(jnp.float32) - float(_N)) / float(_N)
    return masked_delta, prop
